# baseline (speedup 1.0000x reference)
import jax
import jax.numpy as jnp
from jax.experimental import pallas as pl
from jax.experimental.pallas import tpu as pltpu

EPS = 1e-5
EB = 8


def _conv_kernel(xt_ref, w1_ref, w2_ref, t2_ref, o_ref):
    W = o_ref.shape[2]
    C2 = o_ref.shape[1]
    L = W // EB

    h1 = jnp.dot(w1_ref[...], xt_ref[0], preferred_element_type=jnp.float32)
    h1 = jnp.maximum(h1, 0.0).astype(jnp.bfloat16)

    p = jnp.dot(w2_ref[...], h1, preferred_element_type=jnp.float32)
    p0 = p[0:C2]
    p1 = p[C2:2 * C2]
    p2 = p[2 * C2:3 * C2]

    lane = jax.lax.broadcasted_iota(jnp.int32, (1, W), 1) % L
    z = jnp.zeros((C2, 1), jnp.float32)
    sr = jnp.concatenate([z, p0[:, :W - 1]], axis=1)
    sl = jnp.concatenate([p2[:, 1:], z], axis=1)
    acc = p1
    acc = acc + jnp.where(lane == 0, 0.0, sr)
    acc = acc + jnp.where(lane == L - 1, 0.0, sl)
    o = jnp.maximum(acc + t2_ref[...], 0.0)
    o_ref[0] = o.astype(jnp.bfloat16)


def _fc1_kernel(x_ref, w3_ref, o_ref, acc_ref):
    k = pl.program_id(1)
    TC = x_ref.shape[1]
    L = w3_ref.shape[1] // TC

    @pl.when(k == 0)
    def _():
        acc_ref[...] = jnp.zeros_like(acc_ref)

    for c in range(TC):
        w3c = w3_ref[:, c * L:(c + 1) * L].astype(jnp.bfloat16)
        for e in range(EB):
            xce = x_ref[:, c, e * L:(e + 1) * L]
            acc_ref[e] += jax.lax.dot_general(
                xce, w3c,
                dimension_numbers=(((1,), (1,)), ((), ())),
                preferred_element_type=jnp.float32)

    @pl.when(k == pl.num_programs(1) - 1)
    def _():
        o_ref[0] = acc_ref[...]


def _head_kernel(p_ref, s3_ref, t3_ref, w4_ref, b4_ref, o_ref):
    EBd, NB, H = p_ref.shape[1], p_ref.shape[2], p_ref.shape[3]
    CLS = o_ref.shape[1]
    h = (p_ref[0] + p_ref[1]).reshape(EBd * NB, H)
    h3 = jnp.maximum(h * s3_ref[...] + t3_ref[...], 0.0)
    y = jnp.dot(h3, w4_ref[...], preferred_element_type=jnp.float32)
    y = jnp.maximum(y + b4_ref[...], 0.0)
    o_ref[...] = (y.reshape(EBd, NB, CLS).swapaxes(0, 1)
                  .reshape(EBd * NB, CLS))


def kernel(x, w1, b1, g1, be1, w2, b2, g2, be2, w3, b3, g3, be3, w4, b4):
    B, Cin, L = x.shape
    C1 = w1.shape[0]
    C2 = w2.shape[0]
    H = w3.shape[0]
    CLS = w4.shape[0]
    NB = B // EB

    s1 = g1 / jnp.sqrt(1.0 + EPS)
    t1 = b1 * s1 + be1
    s2 = g2 / jnp.sqrt(1.0 + EPS)
    t2 = (b2 * s2 + be2).reshape(-1, 1)
    s3c = (g3 / jnp.sqrt(1.0 + EPS)).reshape(1, -1)
    t3c = (b3 * s3c[0] + be3).reshape(1, -1)

    w1t = jnp.transpose(w1, (2, 1, 0)).reshape(3 * Cin, C1).T * s1[:, None]
    w1s = jnp.concatenate([w1t, t1[:, None]], axis=1)
    w2t = jnp.transpose(w2, (2, 0, 1)).reshape(3 * C2, C1)
    w2s = (w2t * jnp.tile(s2, 3)[:, None]).astype(jnp.bfloat16)

    x_pad = jnp.pad(x, ((0, 0), (0, 0), (1, 1)))
    xt = jnp.stack([x_pad[:, 0, 0:L], x_pad[:, 1, 0:L],
                    x_pad[:, 0, 1:L + 1], x_pad[:, 1, 1:L + 1],
                    x_pad[:, 0, 2:L + 2], x_pad[:, 1, 2:L + 2],
                    jnp.ones((B, L), jnp.float32)], axis=1)
    xt = (xt.reshape(NB, EB, 7, L).transpose(0, 2, 1, 3)
          .reshape(NB, 7, EB * L))

    h2t = pl.pallas_call(
        _conv_kernel,
        out_shape=jax.ShapeDtypeStruct((NB, C2, EB * L), jnp.bfloat16),
        grid=(NB,),
        in_specs=[
            pl.BlockSpec((1, 7, EB * L), lambda b: (b, 0, 0)),
            pl.BlockSpec((C1, 7), lambda b: (0, 0)),
            pl.BlockSpec((3 * C2, C1), lambda b: (0, 0)),
            pl.BlockSpec((C2, 1), lambda b: (0, 0)),
        ],
        out_specs=pl.BlockSpec((1, C2, EB * L), lambda b: (b, 0, 0)),
        compiler_params=pltpu.CompilerParams(
            dimension_semantics=("parallel",)),
    )(xt, w1s, w2s, t2)

    TC = 8
    nk = C2 // TC

    partials = pl.pallas_call(
        _fc1_kernel,
        out_shape=jax.ShapeDtypeStruct((2, EB, NB, H), jnp.float32),
        grid=(2, nk // 2),
        in_specs=[
            pl.BlockSpec((NB, TC, EB * L),
                         lambda c, k: (0, c * (nk // 2) + k, 0)),
            pl.BlockSpec((H, TC * L), lambda c, k: (0, c * (nk // 2) + k)),
        ],
        out_specs=pl.BlockSpec((1, EB, NB, H), lambda c, k: (c, 0, 0, 0)),
        scratch_shapes=[pltpu.VMEM((EB, NB, H), jnp.float32)],
        compiler_params=pltpu.CompilerParams(
            dimension_semantics=("parallel", "arbitrary")),
    )(h2t, w3)

    return pl.pallas_call(
        _head_kernel,
        out_shape=jax.ShapeDtypeStruct((B, CLS), jnp.float32),
        in_specs=[
            pl.BlockSpec((2, EB, NB, H), lambda: (0, 0, 0, 0)),
            pl.BlockSpec((1, H), lambda: (0, 0)),
            pl.BlockSpec((1, H), lambda: (0, 0)),
            pl.BlockSpec((H, CLS), lambda: (0, 0)),
            pl.BlockSpec((1, CLS), lambda: (0, 0)),
        ],
        out_specs=pl.BlockSpec((B, CLS), lambda: (0, 0)),
    )(partials, s3c, t3c, w4.T, b4.reshape(1, -1))


# speedup vs baseline: 1.3611x; 1.3611x over previous
import jax
import jax.numpy as jnp
from jax.experimental import pallas as pl
from jax.experimental.pallas import tpu as pltpu

EPS = 1e-5
EB = 8


def _conv_kernel(x_ref, w1_ref, w2_ref, t2_ref, o_ref):
    L = o_ref.shape[3]
    C2 = o_ref.shape[1]
    W = EB * L

    x2 = jnp.concatenate([x_ref[e] for e in range(EB)], axis=1)
    lane = jax.lax.broadcasted_iota(jnp.int32, (1, W), 1) % L
    z2 = jnp.zeros((2, 1), jnp.float32)
    xsr = jnp.where(lane == 0, 0.0,
                    jnp.concatenate([z2, x2[:, :W - 1]], axis=1))
    xsl = jnp.where(lane == L - 1, 0.0,
                    jnp.concatenate([x2[:, 1:], z2], axis=1))
    xt = jnp.concatenate([xsr, x2, xsl,
                          jnp.ones((1, W), jnp.float32)], axis=0)

    h1 = jnp.dot(w1_ref[...], xt, preferred_element_type=jnp.float32)
    h1 = jnp.maximum(h1, 0.0)

    p = jnp.dot(w2_ref[...], h1, preferred_element_type=jnp.float32)
    p0 = p[0:C2]
    p1 = p[C2:2 * C2]
    p2 = p[2 * C2:3 * C2]
    z = jnp.zeros((C2, 1), jnp.float32)
    sr = jnp.concatenate([z, p0[:, :W - 1]], axis=1)
    sl = jnp.concatenate([p2[:, 1:], z], axis=1)
    acc = p1
    acc = acc + jnp.where(lane == 0, 0.0, sr)
    acc = acc + jnp.where(lane == L - 1, 0.0, sl)
    o = jnp.maximum(acc + t2_ref[...], 0.0).astype(jnp.bfloat16)
    for e in range(EB):
        o_ref[0, :, e, :] = o[:, e * L:(e + 1) * L]


def _fc1_kernel(x_ref, w3_ref, o_ref, acc_ref):
    k = pl.program_id(1)
    L = x_ref.shape[3]
    TC = x_ref.shape[1]
    B = x_ref.shape[0] * x_ref.shape[2]

    @pl.when(k == 0)
    def _():
        acc_ref[...] = jnp.zeros_like(acc_ref)

    for c in range(TC):
        xc = x_ref[:, c].reshape(B, L)
        acc_ref[...] += jax.lax.dot_general(
            xc, w3_ref[:, c * L:(c + 1) * L].astype(jnp.bfloat16),
            dimension_numbers=(((1,), (1,)), ((), ())),
            preferred_element_type=jnp.float32)

    @pl.when(k == pl.num_programs(1) - 1)
    def _():
        o_ref[0] = acc_ref[...]


def _head_kernel(p_ref, s3_ref, t3_ref, w4_ref, b4_ref, o_ref):
    h = p_ref[0] + p_ref[1]
    h3 = jnp.maximum(h * s3_ref[...] + t3_ref[...], 0.0)
    y = jnp.dot(h3, w4_ref[...], preferred_element_type=jnp.float32)
    o_ref[...] = jnp.maximum(y + b4_ref[...], 0.0)


def kernel(x, w1, b1, g1, be1, w2, b2, g2, be2, w3, b3, g3, be3, w4, b4):
    B, Cin, L = x.shape
    C1 = w1.shape[0]
    C2 = w2.shape[0]
    H = w3.shape[0]
    CLS = w4.shape[0]
    NB = B // EB

    s1 = g1 / jnp.sqrt(1.0 + EPS)
    t1 = b1 * s1 + be1
    s2 = g2 / jnp.sqrt(1.0 + EPS)
    t2 = (b2 * s2 + be2).reshape(-1, 1)
    s3c = (g3 / jnp.sqrt(1.0 + EPS)).reshape(1, -1)
    t3c = (b3 * s3c[0] + be3).reshape(1, -1)

    w1t = jnp.transpose(w1, (2, 1, 0)).reshape(3 * Cin, C1).T * s1[:, None]
    w1s = jnp.concatenate([w1t, t1[:, None]], axis=1)
    w2t = jnp.transpose(w2, (2, 0, 1)).reshape(3 * C2, C1)
    w2s = w2t * jnp.tile(s2, 3)[:, None]

    h2t = pl.pallas_call(
        _conv_kernel,
        out_shape=jax.ShapeDtypeStruct((NB, C2, EB, L), jnp.bfloat16),
        grid=(NB,),
        in_specs=[
            pl.BlockSpec((EB, Cin, L), lambda b: (b, 0, 0)),
            pl.BlockSpec((C1, 7), lambda b: (0, 0)),
            pl.BlockSpec((3 * C2, C1), lambda b: (0, 0)),
            pl.BlockSpec((C2, 1), lambda b: (0, 0)),
        ],
        out_specs=pl.BlockSpec((1, C2, EB, L), lambda b: (b, 0, 0, 0)),
        compiler_params=pltpu.CompilerParams(
            dimension_semantics=("parallel",)),
    )(x, w1s, w2s, t2)

    TC = 8
    nk = C2 // TC

    partials = pl.pallas_call(
        _fc1_kernel,
        out_shape=jax.ShapeDtypeStruct((2, B, H), jnp.float32),
        grid=(2, nk // 2),
        in_specs=[
            pl.BlockSpec((NB, TC, EB, L),
                         lambda c, k: (0, c * (nk // 2) + k, 0, 0)),
            pl.BlockSpec((H, TC * L), lambda c, k: (0, c * (nk // 2) + k)),
        ],
        out_specs=pl.BlockSpec((1, B, H), lambda c, k: (c, 0, 0)),
        scratch_shapes=[pltpu.VMEM((B, H), jnp.float32)],
        compiler_params=pltpu.CompilerParams(
            dimension_semantics=("parallel", "arbitrary")),
    )(h2t, w3)

    return pl.pallas_call(
        _head_kernel,
        out_shape=jax.ShapeDtypeStruct((B, CLS), jnp.float32),
        in_specs=[
            pl.BlockSpec((2, B, H), lambda: (0, 0, 0)),
            pl.BlockSpec((1, H), lambda: (0, 0)),
            pl.BlockSpec((1, H), lambda: (0, 0)),
            pl.BlockSpec((H, CLS), lambda: (0, 0)),
            pl.BlockSpec((1, CLS), lambda: (0, 0)),
        ],
        out_specs=pl.BlockSpec((B, CLS), lambda: (0, 0)),
    )(partials, s3c, t3c, w4.T, b4.reshape(1, -1))
